# Initial kernel scaffold; baseline (speedup 1.0000x reference)
import jax
import jax.numpy as jnp
from jax.experimental import pallas as pl


def kernel(h, net_feat, W1, b1, W2, b2, edge_index, cur_nodes, eids, targets, level_id):
    raise NotImplementedError("write your pallas kernel here")


# speedup vs baseline: 6.3994x; 6.3994x over previous
import functools
import jax
import jax.numpy as jnp
from jax import lax
from jax.experimental import pallas as pl
from jax.experimental.pallas import tpu as pltpu
from jax.experimental.pallas import tpu_sc as plsc

N = 10000
E = 320000
D = 128
HID = 256
T = 2000
TP = 2048

NC, NS, L = 2, 16, 16
NW = NC * NS
EPW = E // NW
CH = 80
NCHUNK = EPW // CH
NP = 10240
RPW = NP // NS
CPW = NP // NS

_mesh = plsc.VectorSubcoreMesh(core_axis_name="c", subcore_axis_name="s")
_params = pltpu.CompilerParams(needs_layout_passes=False)


@functools.partial(
    pl.kernel,
    out_type=(
        jax.ShapeDtypeStruct((NP, D), jnp.float32),
        jax.ShapeDtypeStruct((NP, D), jnp.float32),
        jax.ShapeDtypeStruct((NP,), jnp.float32),
        jax.ShapeDtypeStruct((NP,), jnp.float32),
    ),
    mesh=_mesh,
    scratch_types=[
        pltpu.VMEM((CH,), jnp.int32),
        pltpu.VMEM((CH,), jnp.int32),
        pltpu.VMEM((CH, D), jnp.float32),
        pltpu.VMEM((NP,), jnp.float32),
        pltpu.VMEM((CPW,), jnp.float32),
        pltpu.VMEM_SHARED((NP, D), jnp.float32),
        pltpu.VMEM_SHARED((NS, NP), jnp.float32),
        pltpu.SemaphoreType.DMA,
    ],
    compiler_params=_params,
)
def _edge_kernel(h_hbm, src_hbm, dst_hbm, s0_hbm, s1_hbm, c0_hbm, c1_hbm,
                 src_v, dst_v, rows_v, hist_v, chist_v,
                 sums_sp, hist_sp, sem):
    cid = lax.axis_index("c")
    sid = lax.axis_index("s")
    wid = sid * NC + cid

    zero = jnp.zeros((L,), jnp.float32)
    one = jnp.ones((L,), jnp.float32)

    def fill_z(i, _):
        for k in range(D // L):
            rows_v[i, pl.ds(k * L, L)] = zero
        return 0
    lax.fori_loop(0, CH, fill_z, 0)

    base = sid * RPW
    for k in range(RPW // CH):
        pltpu.sync_copy(rows_v, sums_sp.at[pl.ds(base + k * CH, CH)])

    def fill_h(i, _):
        hist_v[pl.ds(i * L, L)] = zero
        return 0
    lax.fori_loop(0, NP // L, fill_h, 0)
    plsc.subcore_barrier()

    estart = wid * EPW

    def edge_step(ci, _):
        off = estart + ci * CH
        pltpu.sync_copy(src_hbm.at[pl.ds(off, CH)], src_v)
        pltpu.sync_copy(dst_hbm.at[pl.ds(off, CH)], dst_v)
        pltpu.async_copy(h_hbm.at[src_v], rows_v, sem).wait()
        pltpu.sync_copy(rows_v, sums_sp.at[dst_v], add=True)
        for g in range(CH // L):
            dvec = dst_v[pl.ds(g * L, L)]
            plsc.addupdate_scatter(hist_v, [dvec], one)
        return 0
    lax.fori_loop(0, NCHUNK, edge_step, 0)

    pltpu.sync_copy(hist_v, hist_sp.at[sid])
    plsc.subcore_barrier()

    cbase = sid * CPW
    for r in range(NS):
        pltpu.sync_copy(hist_sp.at[r, pl.ds(cbase, CPW)],
                        hist_v.at[pl.ds(r * CPW, CPW)])
    for c in range(CPW // L):
        acc = hist_v[pl.ds(c * L, L)]
        for r in range(1, NS):
            acc = acc + hist_v[pl.ds(r * CPW + c * L, L)]
        chist_v[pl.ds(c * L, L)] = acc

    @pl.when(cid == 0)
    def _():
        pltpu.sync_copy(chist_v, c0_hbm.at[pl.ds(cbase, CPW)])
        for k in range(RPW // CH):
            sl = pl.ds(base + k * CH, CH)
            pltpu.sync_copy(sums_sp.at[sl], rows_v)
            pltpu.sync_copy(rows_v, s0_hbm.at[sl])

    @pl.when(cid == 1)
    def _():
        pltpu.sync_copy(chist_v, c1_hbm.at[pl.ds(cbase, CPW)])
        for k in range(RPW // CH):
            sl = pl.ds(base + k * CH, CH)
            pltpu.sync_copy(sums_sp.at[sl], rows_v)
            pltpu.sync_copy(rows_v, s1_hbm.at[sl])


TW = TP // NW


@functools.partial(
    pl.kernel,
    out_type=jax.ShapeDtypeStruct((TP, D), jnp.float32),
    mesh=_mesh,
    scratch_types=[
        pltpu.VMEM((TW,), jnp.int32),
        pltpu.VMEM((TW, D), jnp.float32),
        pltpu.VMEM((TW, D), jnp.float32),
        pltpu.VMEM((TW, D), jnp.float32),
        pltpu.VMEM((NP,), jnp.float32),
        pltpu.VMEM((NP,), jnp.float32),
        pltpu.VMEM((TW, D), jnp.float32),
        pltpu.VMEM((TW,), jnp.float32),
        pltpu.SemaphoreType.DMA,
    ],
    compiler_params=_params,
)
def _finalize_kernel(s0_hbm, s1_hbm, c0_hbm, c1_hbm, mlp_hbm, tgt_hbm,
                     out_hbm, tgt_v, s0_v, s1_v, mlp_v, c0_v, c1_v,
                     o_v, den_v, sem):
    cid = lax.axis_index("c")
    sid = lax.axis_index("s")
    wid = sid * NC + cid
    base = wid * TW

    pltpu.sync_copy(tgt_hbm.at[pl.ds(base, TW)], tgt_v)
    pltpu.sync_copy(c0_hbm, c0_v)
    pltpu.sync_copy(c1_hbm, c1_v)
    pltpu.async_copy(s0_hbm.at[tgt_v], s0_v, sem).wait()
    pltpu.async_copy(s1_hbm.at[tgt_v], s1_v, sem).wait()
    pltpu.async_copy(mlp_hbm.at[tgt_v], mlp_v, sem).wait()

    for i in range(TW // L):
        sl = pl.ds(i * L, L)
        tvec = tgt_v[sl]
        cg = plsc.load_gather(c0_v, [tvec]) + plsc.load_gather(c1_v, [tvec])
        den_v[sl] = jnp.maximum(cg, 1.0)

    zero_i = jnp.zeros((L,), jnp.int32)

    def row_step(j, _):
        den = plsc.load_gather(den_v, [zero_i + j])
        for k in range(D // L):
            sl = pl.ds(k * L, L)
            val = (s0_v[j, sl] + s1_v[j, sl]) / den + mlp_v[j, sl]
            o_v[j, sl] = jnp.maximum(val, 0.0)
        return 0
    lax.fori_loop(0, TW, row_step, 0)

    pltpu.sync_copy(o_v, out_hbm.at[pl.ds(base, TW)])


BN = 1000


def _mlp_body(x_ref, w1_ref, b1_ref, w2_ref, b2_ref, o_ref):
    t = jnp.dot(x_ref[...], w1_ref[...], preferred_element_type=jnp.float32)
    t = jnp.maximum(t + b1_ref[...], 0.0)
    o_ref[...] = (
        jnp.dot(t, w2_ref[...], preferred_element_type=jnp.float32)
        + b2_ref[...]
    )


def _mlp(net_feat, W1, b1, W2, b2):
    return pl.pallas_call(
        _mlp_body,
        grid=(N // BN,),
        in_specs=[
            pl.BlockSpec((BN, D), lambda i: (i, 0)),
            pl.BlockSpec((D, HID), lambda i: (0, 0)),
            pl.BlockSpec((1, HID), lambda i: (0, 0)),
            pl.BlockSpec((HID, D), lambda i: (0, 0)),
            pl.BlockSpec((1, D), lambda i: (0, 0)),
        ],
        out_specs=pl.BlockSpec((BN, D), lambda i: (i, 0)),
        out_shape=jax.ShapeDtypeStruct((N, D), jnp.float32),
    )(net_feat, W1, b1.reshape(1, HID), W2, b2.reshape(1, D))


@jax.jit
def _run(h, net_feat, W1, b1, W2, b2, src, dst, targets):
    s0, s1, c0, c1 = _edge_kernel(h, src, dst)
    mlp_out = _mlp(net_feat, W1, b1, W2, b2)
    tgt_pad = jnp.concatenate(
        [targets, jnp.zeros((TP - T,), jnp.int32)])
    out = _finalize_kernel(s0, s1, c0, c1, mlp_out, tgt_pad)
    return out[:T]


def kernel(h, net_feat, W1, b1, W2, b2, edge_index, cur_nodes, eids,
           targets, level_id):
    src = edge_index[0]
    dst = edge_index[1]
    return _run(h, net_feat, W1, b1, W2, b2, src, dst, targets)


# speedup vs baseline: 10.8171x; 1.6903x over previous
import functools
import jax
import jax.numpy as jnp
from jax import lax
from jax.experimental import pallas as pl
from jax.experimental.pallas import tpu as pltpu
from jax.experimental.pallas import tpu_sc as plsc

N = 10000
E = 320000
D = 128
HID = 256
T = 2000
TP = 2048

NC, NS, L = 2, 16, 16
NW = NC * NS
CH = 64
NCHUNK = 156
EPW = NCHUNK * CH
TAIL = (E - NW * EPW) // NW
NP = 10240
RPW = NP // NS

_mesh = plsc.VectorSubcoreMesh(core_axis_name="c", subcore_axis_name="s")
_params = pltpu.CompilerParams(needs_layout_passes=False)


@functools.partial(
    pl.kernel,
    out_type=(
        jax.ShapeDtypeStruct((NP, D), jnp.float32),
        jax.ShapeDtypeStruct((NP, D), jnp.float32),
        jax.ShapeDtypeStruct((NP,), jnp.float32),
        jax.ShapeDtypeStruct((NP,), jnp.float32),
        jax.ShapeDtypeStruct((NS, NP), jnp.float32),
        jax.ShapeDtypeStruct((NS, NP), jnp.float32),
    ),
    mesh=_mesh,
    scratch_types=[
        [pltpu.VMEM((CH,), jnp.int32)] * 2,
        [pltpu.VMEM((CH,), jnp.int32)] * 2,
        [pltpu.VMEM((CH, D), jnp.float32)] * 2,
        pltpu.VMEM((NP,), jnp.float32),
        pltpu.VMEM((RPW,), jnp.float32),
        pltpu.VMEM((TAIL,), jnp.int32),
        pltpu.VMEM_SHARED((NP, D), jnp.float32),
        [pltpu.SemaphoreType.DMA] * 2,
        [pltpu.SemaphoreType.DMA] * 2,
        pltpu.SemaphoreType.DMA,
    ],
    compiler_params=_params,
)
def _edge_kernel(h_hbm, src_hbm, dst_hbm, s0_hbm, s1_hbm, c0_hbm, c1_hbm,
                 h0_hbm, h1_hbm, src_v, dst_v, rows_v, hist_v, chist_v,
                 tdst_v, sums_sp, sem_i, sem_g, sem_m):
    cid = lax.axis_index("c")
    sid = lax.axis_index("s")
    wid = sid * NC + cid

    zero = jnp.zeros((L,), jnp.float32)
    one = jnp.ones((L,), jnp.float32)

    def fill_z(i, _):
        for k in range(D // L):
            rows_v[0][i, pl.ds(k * L, L)] = zero
        return 0
    lax.fori_loop(0, CH, fill_z, 0)

    base = sid * RPW
    for k in range(RPW // CH):
        pltpu.async_copy(rows_v[0], sums_sp.at[pl.ds(base + k * CH, CH)],
                         sem_m)
    for k in range(RPW // CH):
        pltpu.make_async_copy(rows_v[0],
                              sums_sp.at[pl.ds(base + k * CH, CH)],
                              sem_m).wait()

    def fill_h(i, _):
        hist_v[pl.ds(i * L, L)] = zero
        return 0
    lax.fori_loop(0, NP // L, fill_h, 0)
    plsc.subcore_barrier()

    estart = wid * EPW

    def idx_load(ci, b):
        off = estart + ci * CH
        pltpu.async_copy(src_hbm.at[pl.ds(off, CH)], src_v[b], sem_i[b])
        pltpu.async_copy(dst_hbm.at[pl.ds(off, CH)], dst_v[b], sem_i[b])

    def idx_wait(b):
        pltpu.make_async_copy(src_hbm.at[pl.ds(0, CH)], src_v[b],
                              sem_i[b]).wait()
        pltpu.make_async_copy(dst_hbm.at[pl.ds(0, CH)], dst_v[b],
                              sem_i[b]).wait()

    idx_load(0, 0)
    idx_wait(0)
    idx_load(1, 1)
    pltpu.async_copy(h_hbm.at[src_v[0]], rows_v[0], sem_g[0])

    def outer(go, _):
        for b in range(2):
            ci = 2 * go + b
            nb = 1 - b

            @pl.when(ci + 1 < NCHUNK)
            def _():
                idx_wait(nb)
                pltpu.async_copy(h_hbm.at[src_v[nb]], rows_v[nb], sem_g[nb])

            pltpu.make_async_copy(h_hbm.at[src_v[b]], rows_v[b],
                                  sem_g[b]).wait()
            pltpu.sync_copy(rows_v[b], sums_sp.at[dst_v[b]], add=True)
            for g in range(CH // L):
                dvec = dst_v[b][pl.ds(g * L, L)]
                plsc.addupdate_scatter(hist_v, [dvec], one)

            @pl.when(ci + 2 < NCHUNK)
            def _():
                idx_load(ci + 2, b)
        return 0
    lax.fori_loop(0, NCHUNK // 2, outer, 0)

    toff = NW * EPW + wid * TAIL
    pltpu.sync_copy(src_hbm.at[pl.ds(toff, TAIL)],
                    src_v[0].at[pl.ds(0, TAIL)])
    pltpu.sync_copy(dst_hbm.at[pl.ds(toff, TAIL)], tdst_v)
    pltpu.async_copy(h_hbm.at[src_v[0].at[pl.ds(0, TAIL)]],
                     rows_v[0].at[pl.ds(0, TAIL)], sem_m).wait()
    pltpu.sync_copy(rows_v[0].at[pl.ds(0, TAIL)], sums_sp.at[tdst_v],
                    add=True)
    plsc.addupdate_scatter(hist_v, [tdst_v[...]], one)

    @pl.when(cid == 0)
    def _():
        pltpu.sync_copy(hist_v, h0_hbm.at[sid])

    @pl.when(cid == 1)
    def _():
        pltpu.sync_copy(hist_v, h1_hbm.at[sid])

    plsc.subcore_barrier()

    def merge(hx_hbm):
        for r in range(NS):
            pltpu.async_copy(hx_hbm.at[r, pl.ds(base, RPW)],
                             hist_v.at[pl.ds(r * RPW, RPW)], sem_m)
        for r in range(NS):
            pltpu.make_async_copy(hx_hbm.at[r, pl.ds(base, RPW)],
                                  hist_v.at[pl.ds(r * RPW, RPW)],
                                  sem_m).wait()

    @pl.when(cid == 0)
    def _():
        merge(h0_hbm)

    @pl.when(cid == 1)
    def _():
        merge(h1_hbm)

    for k in range(RPW // L):
        acc = hist_v[pl.ds(k * L, L)]
        for r in range(1, NS):
            acc = acc + hist_v[pl.ds(r * RPW + k * L, L)]
        chist_v[pl.ds(k * L, L)] = acc

    def dump(s_hbm, c_hbm):
        pltpu.sync_copy(chist_v, c_hbm.at[pl.ds(base, RPW)])
        nk = RPW // CH
        for k in range(nk):
            b = k % 2
            sl = pl.ds(base + k * CH, CH)
            if k >= 2:
                psl = pl.ds(base + (k - 2) * CH, CH)
                pltpu.make_async_copy(rows_v[b], s_hbm.at[psl],
                                      sem_g[b]).wait()
            pltpu.sync_copy(sums_sp.at[sl], rows_v[b])
            pltpu.async_copy(rows_v[b], s_hbm.at[sl], sem_g[b])
        for k in (nk - 2, nk - 1):
            b = k % 2
            sl = pl.ds(base + k * CH, CH)
            pltpu.make_async_copy(rows_v[b], s_hbm.at[sl], sem_g[b]).wait()

    @pl.when(cid == 0)
    def _():
        dump(s0_hbm, c0_hbm)

    @pl.when(cid == 1)
    def _():
        dump(s1_hbm, c1_hbm)


TW = TP // NW


@functools.partial(
    pl.kernel,
    out_type=jax.ShapeDtypeStruct((TP, D), jnp.float32),
    mesh=_mesh,
    scratch_types=[
        pltpu.VMEM((TW,), jnp.int32),
        pltpu.VMEM((TW, D), jnp.float32),
        pltpu.VMEM((TW, D), jnp.float32),
        pltpu.VMEM((TW, D), jnp.float32),
        pltpu.VMEM((NP,), jnp.float32),
        pltpu.VMEM((NP,), jnp.float32),
        pltpu.VMEM((TW, D), jnp.float32),
        pltpu.VMEM((TW,), jnp.float32),
        pltpu.SemaphoreType.DMA,
    ],
    compiler_params=_params,
)
def _finalize_kernel(s0_hbm, s1_hbm, c0_hbm, c1_hbm, mlp_hbm, tgt_hbm,
                     out_hbm, tgt_v, s0_v, s1_v, mlp_v, c0_v, c1_v,
                     o_v, den_v, sem):
    cid = lax.axis_index("c")
    sid = lax.axis_index("s")
    wid = sid * NC + cid
    base = wid * TW

    pltpu.sync_copy(tgt_hbm.at[pl.ds(base, TW)], tgt_v)
    pltpu.sync_copy(c0_hbm, c0_v)
    pltpu.sync_copy(c1_hbm, c1_v)
    pltpu.async_copy(s0_hbm.at[tgt_v], s0_v, sem).wait()
    pltpu.async_copy(s1_hbm.at[tgt_v], s1_v, sem).wait()
    pltpu.async_copy(mlp_hbm.at[tgt_v], mlp_v, sem).wait()

    for i in range(TW // L):
        sl = pl.ds(i * L, L)
        tvec = tgt_v[sl]
        cg = plsc.load_gather(c0_v, [tvec]) + plsc.load_gather(c1_v, [tvec])
        den_v[sl] = jnp.maximum(cg, 1.0)

    zero_i = jnp.zeros((L,), jnp.int32)

    def row_step(j, _):
        den = plsc.load_gather(den_v, [zero_i + j])
        for k in range(D // L):
            sl = pl.ds(k * L, L)
            val = (s0_v[j, sl] + s1_v[j, sl]) / den + mlp_v[j, sl]
            o_v[j, sl] = jnp.maximum(val, 0.0)
        return 0
    lax.fori_loop(0, TW, row_step, 0)

    pltpu.sync_copy(o_v, out_hbm.at[pl.ds(base, TW)])


BN = 1000


def _mlp_body(x_ref, w1_ref, b1_ref, w2_ref, b2_ref, o_ref):
    t = jnp.dot(x_ref[...], w1_ref[...], preferred_element_type=jnp.float32)
    t = jnp.maximum(t + b1_ref[...], 0.0)
    o_ref[...] = (
        jnp.dot(t, w2_ref[...], preferred_element_type=jnp.float32)
        + b2_ref[...]
    )


def _mlp(net_feat, W1, b1, W2, b2):
    return pl.pallas_call(
        _mlp_body,
        grid=(N // BN,),
        in_specs=[
            pl.BlockSpec((BN, D), lambda i: (i, 0)),
            pl.BlockSpec((D, HID), lambda i: (0, 0)),
            pl.BlockSpec((1, HID), lambda i: (0, 0)),
            pl.BlockSpec((HID, D), lambda i: (0, 0)),
            pl.BlockSpec((1, D), lambda i: (0, 0)),
        ],
        out_specs=pl.BlockSpec((BN, D), lambda i: (i, 0)),
        out_shape=jax.ShapeDtypeStruct((N, D), jnp.float32),
    )(net_feat, W1, b1.reshape(1, HID), W2, b2.reshape(1, D))


@jax.jit
def _run(h, net_feat, W1, b1, W2, b2, src, dst, targets):
    s0, s1, c0, c1, _, _ = _edge_kernel(h, src, dst)
    mlp_out = _mlp(net_feat, W1, b1, W2, b2)
    tgt_pad = jnp.concatenate(
        [targets, jnp.zeros((TP - T,), jnp.int32)])
    out = _finalize_kernel(s0, s1, c0, c1, mlp_out, tgt_pad)
    return out[:T]


def kernel(h, net_feat, W1, b1, W2, b2, edge_index, cur_nodes, eids,
           targets, level_id):
    src = edge_index[0]
    dst = edge_index[1]
    return _run(h, net_feat, W1, b1, W2, b2, src, dst, targets)


# speedup vs baseline: 13.3412x; 1.2333x over previous
import functools
import jax
import jax.numpy as jnp
from jax import lax
from jax.experimental import pallas as pl
from jax.experimental.pallas import tpu as pltpu
from jax.experimental.pallas import tpu_sc as plsc

N = 10000
E = 320000
D = 128
HID = 256
T = 2000
TP = 2048

NC, NS, L = 2, 16, 16
NW = NC * NS
CH = 64
NCHUNK = 156
EPW = NCHUNK * CH
TAIL = (E - NW * EPW) // NW
NP = 10240
RPW = NP // NS

_mesh = plsc.VectorSubcoreMesh(core_axis_name="c", subcore_axis_name="s")
_params = pltpu.CompilerParams(needs_layout_passes=False)


@functools.partial(
    pl.kernel,
    out_type=(
        jax.ShapeDtypeStruct((NP, D), jnp.float32),
        jax.ShapeDtypeStruct((NP, D), jnp.float32),
        jax.ShapeDtypeStruct((NP,), jnp.float32),
        jax.ShapeDtypeStruct((NP,), jnp.float32),
        jax.ShapeDtypeStruct((NS, NP), jnp.float32),
        jax.ShapeDtypeStruct((NS, NP), jnp.float32),
    ),
    mesh=_mesh,
    scratch_types=[
        [pltpu.VMEM((CH,), jnp.int32)] * 4,
        [pltpu.VMEM((CH,), jnp.int32)] * 4,
        [pltpu.VMEM((CH, D), jnp.float32)] * 2,
        pltpu.VMEM((NP,), jnp.float32),
        pltpu.VMEM((RPW,), jnp.float32),
        pltpu.VMEM((TAIL,), jnp.int32),
        pltpu.VMEM_SHARED((NP, D), jnp.float32),
        [pltpu.SemaphoreType.DMA] * 4,
        [pltpu.SemaphoreType.DMA] * 2,
        [pltpu.SemaphoreType.DMA] * 2,
        pltpu.SemaphoreType.DMA,
    ],
    compiler_params=_params,
)
def _edge_kernel(h_hbm, src_hbm, dst_hbm, s0_hbm, s1_hbm, c0_hbm, c1_hbm,
                 h0_hbm, h1_hbm, src_v, dst_v, rows_v, hist_v, chist_v,
                 tdst_v, sums_sp, sem_i, sem_g, sem_s, sem_m):
    cid = lax.axis_index("c")
    sid = lax.axis_index("s")
    wid = sid * NC + cid

    zero = jnp.zeros((L,), jnp.float32)
    one = jnp.ones((L,), jnp.float32)

    def fill_z(i, _):
        for k in range(D // L):
            rows_v[0][i, pl.ds(k * L, L)] = zero
        return 0
    lax.fori_loop(0, CH, fill_z, 0)

    base = sid * RPW
    for k in range(RPW // CH):
        pltpu.async_copy(rows_v[0], sums_sp.at[pl.ds(base + k * CH, CH)],
                         sem_m)
    for k in range(RPW // CH):
        pltpu.make_async_copy(rows_v[0],
                              sums_sp.at[pl.ds(base + k * CH, CH)],
                              sem_m).wait()

    def fill_h(i, _):
        hist_v[pl.ds(i * L, L)] = zero
        return 0
    lax.fori_loop(0, NP // L, fill_h, 0)
    plsc.subcore_barrier()

    estart = wid * EPW

    def idx_load(ci, b):
        off = estart + ci * CH
        pltpu.async_copy(src_hbm.at[pl.ds(off, CH)], src_v[b], sem_i[b])
        pltpu.async_copy(dst_hbm.at[pl.ds(off, CH)], dst_v[b], sem_i[b])

    def idx_wait(b):
        pltpu.make_async_copy(src_hbm.at[pl.ds(0, CH)], src_v[b],
                              sem_i[b]).wait()
        pltpu.make_async_copy(dst_hbm.at[pl.ds(0, CH)], dst_v[b],
                              sem_i[b]).wait()

    def scat_wait(b):
        pltpu.make_async_copy(rows_v[b], sums_sp.at[dst_v[0]],
                              sem_s[b]).wait()

    for q in range(3):
        idx_load(q, q)
    idx_wait(0)
    pltpu.async_copy(h_hbm.at[src_v[0]], rows_v[0], sem_g[0])

    def outer(go, _):
        for u in range(4):
            ci = 4 * go + u
            b = u % 2
            nb = 1 - b
            ib = u
            nib = (u + 1) % 4

            if u == 0:
                @pl.when(go >= 1)
                def _():
                    scat_wait(nb)
            else:
                scat_wait(nb)

            @pl.when(ci + 1 < NCHUNK)
            def _():
                idx_wait(nib)
                pltpu.async_copy(h_hbm.at[src_v[nib]], rows_v[nb], sem_g[nb])

            pltpu.make_async_copy(h_hbm.at[src_v[ib]], rows_v[b],
                                  sem_g[b]).wait()
            pltpu.async_copy(rows_v[b], sums_sp.at[dst_v[ib]], sem_s[b],
                             add=True)
            for g in range(CH // L):
                dvec = dst_v[ib][pl.ds(g * L, L)]
                plsc.addupdate_scatter(hist_v, [dvec], one)

            @pl.when(ci + 3 < NCHUNK)
            def _():
                idx_load(ci + 3, (u + 3) % 4)
        return 0
    lax.fori_loop(0, NCHUNK // 4, outer, 0)
    scat_wait((NCHUNK - 1) % 2)

    toff = NW * EPW + wid * TAIL
    pltpu.sync_copy(src_hbm.at[pl.ds(toff, TAIL)],
                    src_v[0].at[pl.ds(0, TAIL)])
    pltpu.sync_copy(dst_hbm.at[pl.ds(toff, TAIL)], tdst_v)
    pltpu.async_copy(h_hbm.at[src_v[0].at[pl.ds(0, TAIL)]],
                     rows_v[0].at[pl.ds(0, TAIL)], sem_m).wait()
    pltpu.sync_copy(rows_v[0].at[pl.ds(0, TAIL)], sums_sp.at[tdst_v],
                    add=True)
    plsc.addupdate_scatter(hist_v, [tdst_v[...]], one)

    @pl.when(cid == 0)
    def _():
        pltpu.sync_copy(hist_v, h0_hbm.at[sid])

    @pl.when(cid == 1)
    def _():
        pltpu.sync_copy(hist_v, h1_hbm.at[sid])

    plsc.subcore_barrier()

    def merge(hx_hbm):
        for r in range(NS):
            pltpu.async_copy(hx_hbm.at[r, pl.ds(base, RPW)],
                             hist_v.at[pl.ds(r * RPW, RPW)], sem_m)
        for r in range(NS):
            pltpu.make_async_copy(hx_hbm.at[r, pl.ds(base, RPW)],
                                  hist_v.at[pl.ds(r * RPW, RPW)],
                                  sem_m).wait()

    @pl.when(cid == 0)
    def _():
        merge(h0_hbm)

    @pl.when(cid == 1)
    def _():
        merge(h1_hbm)

    for k in range(RPW // L):
        acc = hist_v[pl.ds(k * L, L)]
        for r in range(1, NS):
            acc = acc + hist_v[pl.ds(r * RPW + k * L, L)]
        chist_v[pl.ds(k * L, L)] = acc

    def dump(s_hbm, c_hbm):
        pltpu.sync_copy(chist_v, c_hbm.at[pl.ds(base, RPW)])
        nk = RPW // CH
        for k in range(nk):
            b = k % 2
            sl = pl.ds(base + k * CH, CH)
            if k >= 2:
                psl = pl.ds(base + (k - 2) * CH, CH)
                pltpu.make_async_copy(rows_v[b], s_hbm.at[psl],
                                      sem_g[b]).wait()
            pltpu.sync_copy(sums_sp.at[sl], rows_v[b])
            pltpu.async_copy(rows_v[b], s_hbm.at[sl], sem_g[b])
        for k in (nk - 2, nk - 1):
            b = k % 2
            sl = pl.ds(base + k * CH, CH)
            pltpu.make_async_copy(rows_v[b], s_hbm.at[sl], sem_g[b]).wait()

    @pl.when(cid == 0)
    def _():
        dump(s0_hbm, c0_hbm)

    @pl.when(cid == 1)
    def _():
        dump(s1_hbm, c1_hbm)


TW = TP // NW


@functools.partial(
    pl.kernel,
    out_type=jax.ShapeDtypeStruct((TP, D), jnp.float32),
    mesh=_mesh,
    scratch_types=[
        pltpu.VMEM((TW,), jnp.int32),
        pltpu.VMEM((TW, D), jnp.float32),
        pltpu.VMEM((TW, D), jnp.float32),
        pltpu.VMEM((TW, D), jnp.float32),
        pltpu.VMEM((NP,), jnp.float32),
        pltpu.VMEM((NP,), jnp.float32),
        pltpu.VMEM((TW, D), jnp.float32),
        pltpu.VMEM((TW,), jnp.float32),
        pltpu.SemaphoreType.DMA,
    ],
    compiler_params=_params,
)
def _finalize_kernel(s0_hbm, s1_hbm, c0_hbm, c1_hbm, mlp_hbm, tgt_hbm,
                     out_hbm, tgt_v, s0_v, s1_v, mlp_v, c0_v, c1_v,
                     o_v, den_v, sem):
    cid = lax.axis_index("c")
    sid = lax.axis_index("s")
    wid = sid * NC + cid
    base = wid * TW

    pltpu.sync_copy(tgt_hbm.at[pl.ds(base, TW)], tgt_v)
    pltpu.async_copy(c0_hbm, c0_v, sem)
    pltpu.async_copy(c1_hbm, c1_v, sem)
    pltpu.async_copy(s0_hbm.at[tgt_v], s0_v, sem)
    pltpu.async_copy(s1_hbm.at[tgt_v], s1_v, sem)
    pltpu.async_copy(mlp_hbm.at[tgt_v], mlp_v, sem)
    pltpu.make_async_copy(c0_hbm, c0_v, sem).wait()
    pltpu.make_async_copy(c1_hbm, c1_v, sem).wait()
    pltpu.make_async_copy(s0_hbm.at[tgt_v], s0_v, sem).wait()
    pltpu.make_async_copy(s1_hbm.at[tgt_v], s1_v, sem).wait()
    pltpu.make_async_copy(mlp_hbm.at[tgt_v], mlp_v, sem).wait()

    for i in range(TW // L):
        sl = pl.ds(i * L, L)
        tvec = tgt_v[sl]
        cg = plsc.load_gather(c0_v, [tvec]) + plsc.load_gather(c1_v, [tvec])
        den_v[sl] = jnp.maximum(cg, 1.0)

    zero_i = jnp.zeros((L,), jnp.int32)

    def row_step(j, _):
        den = plsc.load_gather(den_v, [zero_i + j])
        for k in range(D // L):
            sl = pl.ds(k * L, L)
            val = (s0_v[j, sl] + s1_v[j, sl]) / den + mlp_v[j, sl]
            o_v[j, sl] = jnp.maximum(val, 0.0)
        return 0
    lax.fori_loop(0, TW, row_step, 0)

    pltpu.sync_copy(o_v, out_hbm.at[pl.ds(base, TW)])


BN = 1000


def _mlp_body(x_ref, w1_ref, b1_ref, w2_ref, b2_ref, o_ref):
    t = jnp.dot(x_ref[...], w1_ref[...], preferred_element_type=jnp.float32)
    t = jnp.maximum(t + b1_ref[...], 0.0)
    o_ref[...] = (
        jnp.dot(t, w2_ref[...], preferred_element_type=jnp.float32)
        + b2_ref[...]
    )


def _mlp(net_feat, W1, b1, W2, b2):
    return pl.pallas_call(
        _mlp_body,
        grid=(N // BN,),
        in_specs=[
            pl.BlockSpec((BN, D), lambda i: (i, 0)),
            pl.BlockSpec((D, HID), lambda i: (0, 0)),
            pl.BlockSpec((1, HID), lambda i: (0, 0)),
            pl.BlockSpec((HID, D), lambda i: (0, 0)),
            pl.BlockSpec((1, D), lambda i: (0, 0)),
        ],
        out_specs=pl.BlockSpec((BN, D), lambda i: (i, 0)),
        out_shape=jax.ShapeDtypeStruct((N, D), jnp.float32),
    )(net_feat, W1, b1.reshape(1, HID), W2, b2.reshape(1, D))


@jax.jit
def _run(h, net_feat, W1, b1, W2, b2, src, dst, targets):
    s0, s1, c0, c1, _, _ = _edge_kernel(h, src, dst)
    mlp_out = _mlp(net_feat, W1, b1, W2, b2)
    tgt_pad = jnp.concatenate(
        [targets, jnp.zeros((TP - T,), jnp.int32)])
    out = _finalize_kernel(s0, s1, c0, c1, mlp_out, tgt_pad)
    return out[:T]


def kernel(h, net_feat, W1, b1, W2, b2, edge_index, cur_nodes, eids,
           targets, level_id):
    src = edge_index[0]
    dst = edge_index[1]
    return _run(h, net_feat, W1, b1, W2, b2, src, dst, targets)


# speedup vs baseline: 15.5122x; 1.1627x over previous
import functools
import jax
import jax.numpy as jnp
from jax import lax
from jax.experimental import pallas as pl
from jax.experimental.pallas import tpu as pltpu
from jax.experimental.pallas import tpu_sc as plsc

N = 10000
E = 320000
D = 128
HID = 256
T = 2000
TP = 2048

NC, NS, L = 2, 16, 16
NW = NC * NS
CHR = 64
NRAW = 156
EPW = NRAW * CHR
TAIL = (E - NW * EPW) // NW
NP = 10240
RPW = NP // NS
CD = 56
RING = 4
RLEN = RING * CD
ZCHK = (CD,) * (RPW // CD) + ((RPW % CD,) if RPW % CD else ())
NEG = -1.0e30

_mesh = plsc.VectorSubcoreMesh(core_axis_name="c", subcore_axis_name="s")
_params = pltpu.CompilerParams(needs_layout_passes=False)


@functools.partial(
    pl.kernel,
    out_type=(
        jax.ShapeDtypeStruct((NP, D), jnp.float32),
        jax.ShapeDtypeStruct((NP, D), jnp.float32),
        jax.ShapeDtypeStruct((NP,), jnp.float32),
        jax.ShapeDtypeStruct((NP,), jnp.float32),
        jax.ShapeDtypeStruct((NS, NP), jnp.float32),
        jax.ShapeDtypeStruct((NS, NP), jnp.float32),
    ),
    mesh=_mesh,
    scratch_types=[
        [pltpu.VMEM((CHR,), jnp.int32)] * 2,
        [pltpu.VMEM((CHR,), jnp.int32)] * 2,
        [pltpu.VMEM((CD, D), jnp.float32)] * RING,
        pltpu.VMEM((RLEN,), jnp.int32),
        pltpu.VMEM((RING, CD), jnp.int32),
        pltpu.VMEM((NP,), jnp.float32),
        pltpu.VMEM((TP,), jnp.int32),
        pltpu.VMEM_SHARED((NP, D), jnp.float32),
        [pltpu.SemaphoreType.DMA] * 2,
        [pltpu.SemaphoreType.DMA] * RING,
        [pltpu.SemaphoreType.DMA] * RING,
        pltpu.SemaphoreType.DMA,
    ],
    compiler_params=_params,
)
def _edge_kernel(h_hbm, src_hbm, dst_hbm, tgt_hbm, s0_hbm, s1_hbm, c0_hbm,
                 c1_hbm, h0_hbm, h1_hbm, sraw_v, draw_v, rows_v, csrc_v,
                 cdst_v, hist_v, tg_v, sums_sp, sem_r, sem_g, sem_s, sem_m):
    cid = lax.axis_index("c")
    sid = lax.axis_index("s")
    wid = sid * NC + cid

    zero = jnp.zeros((L,), jnp.float32)
    one = jnp.ones((L,), jnp.float32)
    zerov = jnp.zeros((L,), jnp.float32)
    lanes = lax.iota(jnp.int32, L)

    def fill_z(i, _):
        for k in range(D // L):
            rows_v[0][i, pl.ds(k * L, L)] = zero
        return 0
    lax.fori_loop(0, CD, fill_z, 0)

    base = sid * RPW
    for k, n in enumerate(ZCHK):
        pltpu.async_copy(rows_v[0].at[pl.ds(0, n)],
                         sums_sp.at[pl.ds(base + k * CD, n)], sem_m)
    for k, n in enumerate(ZCHK):
        pltpu.make_async_copy(rows_v[0].at[pl.ds(0, n)],
                              sums_sp.at[pl.ds(base + k * CD, n)],
                              sem_m).wait()

    negv = jnp.full((L,), NEG, jnp.float32)

    def fill_h(i, _):
        hist_v[pl.ds(i * L, L)] = negv
        return 0
    lax.fori_loop(0, NP // L, fill_h, 0)

    pltpu.sync_copy(tgt_hbm, tg_v)
    for i in range(TP // L):
        tv = tg_v[pl.ds(i * L, L)]
        plsc.store_scatter(hist_v, [tv], zerov)
    plsc.subcore_barrier()

    estart = wid * EPW

    def raw_load(r, b):
        off = estart + r * CHR
        pltpu.async_copy(src_hbm.at[pl.ds(off, CHR)], sraw_v[b], sem_r[b])
        pltpu.async_copy(dst_hbm.at[pl.ds(off, CHR)], draw_v[b], sem_r[b])

    def raw_wait(b):
        pltpu.make_async_copy(src_hbm.at[pl.ds(0, CHR)], sraw_v[b],
                              sem_r[b]).wait()
        pltpu.make_async_copy(dst_hbm.at[pl.ds(0, CHR)], draw_v[b],
                              sem_r[b]).wait()

    def gather_wait(s):
        pltpu.make_async_copy(h_hbm.at[csrc_v.at[pl.ds(s * CD, CD)]],
                              rows_v[s], sem_g[s]).wait()

    def scat_issue(s):
        pltpu.async_copy(rows_v[s], sums_sp.at[cdst_v.at[s]], sem_s[s],
                         add=True)

    def scat_wait(s):
        pltpu.make_async_copy(rows_v[s], sums_sp.at[cdst_v.at[s]],
                              sem_s[s]).wait()

    def fire(s, nf):
        os = (s + RING - 1) % RING

        @pl.when(nf >= 1)
        def _():
            gather_wait(os)
            scat_issue(os)

        ps = (s + RING - 2) % RING

        @pl.when(nf >= 2)
        def _():
            scat_wait(ps)

        pltpu.async_copy(h_hbm.at[csrc_v.at[pl.ds(s * CD, CD)]],
                         rows_v[s], sem_g[s])

    def group(svec, dvec, m, carry, update_hist):
        wq, fill, slot, nf = carry
        if update_hist:
            plsc.addupdate_scatter(hist_v, [dvec], one, mask=m)
        mi = jnp.where(m, 1, 0)
        cs = plsc.cumsum(mi)
        cnt = jnp.sum(mi)
        pos = wq + cs - 1
        pos = jnp.where(pos >= RLEN, pos - RLEN, pos)
        slotv = jnp.where(pos >= CD, 1, 0)
        for kk in range(2, RING):
            slotv = slotv + jnp.where(pos >= kk * CD, 1, 0)
        within = pos - slotv * CD
        plsc.store_scatter(csrc_v, [pos], svec, mask=m)
        plsc.store_scatter(cdst_v, [slotv, within], dvec, mask=m)
        wq = wq + cnt
        wq = jnp.where(wq >= RLEN, wq - RLEN, wq)
        fill = fill + cnt
        fcond = fill >= CD
        for s in range(RING):
            @pl.when(fcond & (slot == s))
            def _():
                fire(s, nf)
        fill = jnp.where(fcond, fill - CD, fill)
        slot = jnp.where(fcond, slot + 1, slot)
        slot = jnp.where(slot == RING, 0, slot)
        nf = nf + jnp.where(fcond, 1, 0)
        return wq, fill, slot, nf

    def filt_group(svec, dvec, carry):
        mv = plsc.load_gather(hist_v, [dvec])
        m = mv > (NEG * 0.5)
        return group(svec, dvec, m, carry, True)

    raw_load(0, 0)
    raw_load(1, 1)

    def outer(go, carry):
        for b in range(2):
            r = 2 * go + b
            raw_wait(b)
            for g in range(CHR // L):
                sl = pl.ds(g * L, L)
                carry = filt_group(sraw_v[b][sl], draw_v[b][sl], carry)

            @pl.when(r + 2 < NRAW)
            def _():
                raw_load(r + 2, b)
        return carry

    carry = lax.fori_loop(
        0, NRAW // 2, outer,
        (jnp.int32(0), jnp.int32(0), jnp.int32(0), jnp.int32(0)))

    toff = NW * EPW + wid * TAIL
    pltpu.sync_copy(src_hbm.at[pl.ds(toff, TAIL)],
                    sraw_v[0].at[pl.ds(0, TAIL)])
    pltpu.sync_copy(dst_hbm.at[pl.ds(toff, TAIL)],
                    draw_v[0].at[pl.ds(0, TAIL)])
    carry = filt_group(sraw_v[0][pl.ds(0, TAIL)],
                       draw_v[0][pl.ds(0, TAIL)], carry)

    _, fill0, _, _ = carry
    npad = jnp.where(fill0 > 0, CD - fill0, 0)
    dummy_s = jnp.zeros((L,), jnp.int32)
    dummy_d = jnp.full((L,), NP - 1, jnp.int32)
    for it in range(4):
        m = (lanes + it * L) < npad
        carry = group(dummy_s, dummy_d, m, carry, False)
    _, _, _, nf = carry

    m1 = lax.rem(nf + RING - 1, RING)
    m2 = lax.rem(nf + RING - 2, RING)
    for s in range(RING):
        @pl.when((nf >= 1) & (m1 == s))
        def _():
            gather_wait(s)
            scat_issue(s)
    for s in range(RING):
        @pl.when((nf >= 2) & (m2 == s))
        def _():
            scat_wait(s)
    for s in range(RING):
        @pl.when((nf >= 1) & (m1 == s))
        def _():
            scat_wait(s)

    @pl.when(cid == 0)
    def _():
        pltpu.sync_copy(hist_v, h0_hbm.at[sid])

    @pl.when(cid == 1)
    def _():
        pltpu.sync_copy(hist_v, h1_hbm.at[sid])

    plsc.subcore_barrier()

    def merge(hx_hbm):
        for rr in range(NS):
            pltpu.async_copy(hx_hbm.at[rr, pl.ds(base, RPW)],
                             hist_v.at[pl.ds(rr * RPW, RPW)], sem_m)
        for rr in range(NS):
            pltpu.make_async_copy(hx_hbm.at[rr, pl.ds(base, RPW)],
                                  hist_v.at[pl.ds(rr * RPW, RPW)],
                                  sem_m).wait()

    @pl.when(cid == 0)
    def _():
        merge(h0_hbm)

    @pl.when(cid == 1)
    def _():
        merge(h1_hbm)

    for k in range(RPW // L):
        acc = hist_v[pl.ds(k * L, L)]
        for rr in range(1, NS):
            acc = acc + hist_v[pl.ds(rr * RPW + k * L, L)]
        hist_v[pl.ds(k * L, L)] = acc

    def dump(s_hbm, c_hbm):
        pltpu.sync_copy(hist_v.at[pl.ds(0, RPW)],
                        c_hbm.at[pl.ds(base, RPW)])
        for k, n in enumerate(ZCHK):
            b = k % 2
            sl = pl.ds(base + k * CD, n)
            if k >= 2:
                pn = ZCHK[k - 2]
                psl = pl.ds(base + (k - 2) * CD, pn)
                pltpu.make_async_copy(rows_v[b].at[pl.ds(0, pn)],
                                      s_hbm.at[psl], sem_g[b]).wait()
            pltpu.sync_copy(sums_sp.at[sl], rows_v[b].at[pl.ds(0, n)])
            pltpu.async_copy(rows_v[b].at[pl.ds(0, n)], s_hbm.at[sl],
                             sem_g[b])
        for k in (len(ZCHK) - 2, len(ZCHK) - 1):
            b = k % 2
            n = ZCHK[k]
            sl = pl.ds(base + k * CD, n)
            pltpu.make_async_copy(rows_v[b].at[pl.ds(0, n)], s_hbm.at[sl],
                                  sem_g[b]).wait()

    @pl.when(cid == 0)
    def _():
        dump(s0_hbm, c0_hbm)

    @pl.when(cid == 1)
    def _():
        dump(s1_hbm, c1_hbm)


TW = TP // NW


@functools.partial(
    pl.kernel,
    out_type=jax.ShapeDtypeStruct((TP, D), jnp.float32),
    mesh=_mesh,
    scratch_types=[
        pltpu.VMEM((TW,), jnp.int32),
        pltpu.VMEM((TW, D), jnp.float32),
        pltpu.VMEM((TW, D), jnp.float32),
        pltpu.VMEM((TW, D), jnp.float32),
        pltpu.VMEM((NP,), jnp.float32),
        pltpu.VMEM((NP,), jnp.float32),
        pltpu.VMEM((TW, D), jnp.float32),
        pltpu.VMEM((TW,), jnp.float32),
        pltpu.SemaphoreType.DMA,
    ],
    compiler_params=_params,
)
def _finalize_kernel(s0_hbm, s1_hbm, c0_hbm, c1_hbm, mlp_hbm, tgt_hbm,
                     out_hbm, tgt_v, s0_v, s1_v, mlp_v, c0_v, c1_v,
                     o_v, den_v, sem):
    cid = lax.axis_index("c")
    sid = lax.axis_index("s")
    wid = sid * NC + cid
    base = wid * TW

    pltpu.sync_copy(tgt_hbm.at[pl.ds(base, TW)], tgt_v)
    pltpu.async_copy(c0_hbm, c0_v, sem)
    pltpu.async_copy(c1_hbm, c1_v, sem)
    pltpu.async_copy(s0_hbm.at[tgt_v], s0_v, sem)
    pltpu.async_copy(s1_hbm.at[tgt_v], s1_v, sem)
    pltpu.async_copy(mlp_hbm.at[tgt_v], mlp_v, sem)
    pltpu.make_async_copy(c0_hbm, c0_v, sem).wait()
    pltpu.make_async_copy(c1_hbm, c1_v, sem).wait()
    pltpu.make_async_copy(s0_hbm.at[tgt_v], s0_v, sem).wait()
    pltpu.make_async_copy(s1_hbm.at[tgt_v], s1_v, sem).wait()
    pltpu.make_async_copy(mlp_hbm.at[tgt_v], mlp_v, sem).wait()

    for i in range(TW // L):
        sl = pl.ds(i * L, L)
        tvec = tgt_v[sl]
        cg = plsc.load_gather(c0_v, [tvec]) + plsc.load_gather(c1_v, [tvec])
        den_v[sl] = jnp.maximum(cg, 1.0)

    zero_i = jnp.zeros((L,), jnp.int32)

    def row_step(j, _):
        den = plsc.load_gather(den_v, [zero_i + j])
        for k in range(D // L):
            sl = pl.ds(k * L, L)
            val = (s0_v[j, sl] + s1_v[j, sl]) / den + mlp_v[j, sl]
            o_v[j, sl] = jnp.maximum(val, 0.0)
        return 0
    lax.fori_loop(0, TW, row_step, 0)

    pltpu.sync_copy(o_v, out_hbm.at[pl.ds(base, TW)])


BN = 1000


def _mlp_body(x_ref, w1_ref, b1_ref, w2_ref, b2_ref, o_ref):
    t = jnp.dot(x_ref[...], w1_ref[...], preferred_element_type=jnp.float32)
    t = jnp.maximum(t + b1_ref[...], 0.0)
    o_ref[...] = (
        jnp.dot(t, w2_ref[...], preferred_element_type=jnp.float32)
        + b2_ref[...]
    )


def _mlp(net_feat, W1, b1, W2, b2):
    return pl.pallas_call(
        _mlp_body,
        grid=(N // BN,),
        in_specs=[
            pl.BlockSpec((BN, D), lambda i: (i, 0)),
            pl.BlockSpec((D, HID), lambda i: (0, 0)),
            pl.BlockSpec((1, HID), lambda i: (0, 0)),
            pl.BlockSpec((HID, D), lambda i: (0, 0)),
            pl.BlockSpec((1, D), lambda i: (0, 0)),
        ],
        out_specs=pl.BlockSpec((BN, D), lambda i: (i, 0)),
        out_shape=jax.ShapeDtypeStruct((N, D), jnp.float32),
    )(net_feat, W1, b1.reshape(1, HID), W2, b2.reshape(1, D))


@jax.jit
def _run(h, net_feat, W1, b1, W2, b2, src, dst, targets):
    tgt_pad = jnp.concatenate(
        [targets, jnp.zeros((TP - T,), jnp.int32)])
    s0, s1, c0, c1, _, _ = _edge_kernel(h, src, dst, tgt_pad)
    mlp_out = _mlp(net_feat, W1, b1, W2, b2)
    out = _finalize_kernel(s0, s1, c0, c1, mlp_out, tgt_pad)
    return out[:T]


def kernel(h, net_feat, W1, b1, W2, b2, edge_index, cur_nodes, eids,
           targets, level_id):
    src = edge_index[0]
    dst = edge_index[1]
    return _run(h, net_feat, W1, b1, W2, b2, src, dst, targets)
